# Initial kernel scaffold; baseline (speedup 1.0000x reference)
import jax
import jax.numpy as jnp
from jax.experimental import pallas as pl


def kernel(x, edge_index, W0, b0, W1, b1):
    raise NotImplementedError("write your pallas kernel here")


# speedup vs baseline: 7.1309x; 7.1309x over previous
import functools

import jax
import jax.numpy as jnp
from jax import lax
from jax.experimental import pallas as pl
from jax.experimental.pallas import tpu as pltpu
from jax.experimental.pallas import tpu_sc as plsc

N = 10000
F = 128
H = 128
T = 4
NC = 2
NS = 16
NW = NC * NS
CHUNK = 128
NCHUNK = 80
HC = NCHUNK // 2
EW = CHUNK * NCHUNK
E_PAD = EW * NW
R = 632
N_PAD = R * NS
BLK = 1264
GRID = N_PAD // BLK

_MESH = plsc.VectorSubcoreMesh(
    core_axis_name="c", subcore_axis_name="s", num_cores=NC, num_subcores=NS)


def _hist_body(dst_hbm, ones_hbm, zeros_hbm, out_hbm, dst_v, ones_v, acc):
  c = lax.axis_index("c")
  s = lax.axis_index("s")
  pltpu.sync_copy(dst_hbm.at[c, s], dst_v)
  pltpu.sync_copy(ones_hbm, ones_v)
  pltpu.sync_copy(zeros_hbm.at[pl.ds(s * R, R)], acc.at[pl.ds(s * R, R)])
  plsc.subcore_barrier()

  def body(j, carry):
    pltpu.sync_copy(ones_v, acc.at[dst_v.at[j]], add=True)
    return carry

  lax.fori_loop(0, NCHUNK, body, 0)
  plsc.subcore_barrier()
  pltpu.sync_copy(acc.at[pl.ds(s * R, R)], out_hbm.at[c, pl.ds(s * R, R)])


_hist_call = pl.kernel(
    _hist_body,
    out_type=jax.ShapeDtypeStruct((NC, N_PAD, F), jnp.float32),
    mesh=_MESH,
    scratch_types=[
        pltpu.VMEM((NCHUNK, CHUNK), jnp.int32),
        pltpu.VMEM((CHUNK, F), jnp.float32),
        pltpu.VMEM_SHARED((N_PAD, F), jnp.float32),
    ],
)


def _make_prop(nt):
  def body(table_hbm, src_hbm, dst_hbm, zeros_hbm, out_hbm,
           src_v, dst_v, buf0, buf1, acc, sem0, sem1):
    c = lax.axis_index("c")
    s = lax.axis_index("s")
    pltpu.sync_copy(src_hbm.at[c, s], src_v)
    bufs = (buf0, buf1)
    sems = (sem0, sem1)
    for t in range(nt):
      table = table_hbm.at[t]
      pltpu.sync_copy(zeros_hbm.at[pl.ds(s * R, R)], acc.at[pl.ds(s * R, R)])
      plsc.subcore_barrier()
      for half in range(2):
        base = half * HC
        pltpu.sync_copy(dst_hbm.at[c, s, pl.ds(base, HC)], dst_v)
        pltpu.async_copy(table.at[src_v.at[base]], buf0, sem0)
        pltpu.async_copy(table.at[src_v.at[base + 1]], buf1, sem1)

        def body2(j2, carry):
          for p in range(2):
            j = j2 * 2 + p
            pltpu.make_async_copy(
                table.at[src_v.at[base + j]], bufs[p], sems[p]).wait()
            pltpu.sync_copy(bufs[p], acc.at[dst_v.at[j]], add=True)

            @pl.when(j + 2 < HC)
            def _issue():
              pltpu.async_copy(table.at[src_v.at[base + j + 2]], bufs[p], sems[p])
          return carry

        lax.fori_loop(0, HC // 2, body2, 0)
      plsc.subcore_barrier()
      pltpu.sync_copy(acc.at[pl.ds(s * R, R)], out_hbm.at[t, c, pl.ds(s * R, R)])

  return pl.kernel(
      body,
      out_type=jax.ShapeDtypeStruct((nt, NC, N_PAD, F), jnp.float32),
      mesh=_MESH,
      scratch_types=[
          pltpu.VMEM((NCHUNK, CHUNK), jnp.int32),
          pltpu.VMEM((HC, CHUNK), jnp.int32),
          pltpu.VMEM((CHUNK, F), jnp.float32),
          pltpu.VMEM((CHUNK, F), jnp.float32),
          pltpu.VMEM_SHARED((N_PAD, F), jnp.float32),
          pltpu.SemaphoreType.DMA,
          pltpu.SemaphoreType.DMA,
      ],
  )


_prop1 = _make_prop(1)
_prop4 = _make_prop(T)


def _prep_body(hist_ref, x_ref, dis_ref, ux_ref):
  deg = hist_ref[0, :, 0:1] + hist_ref[1, :, 0:1] + 1.0
  d = lax.rsqrt(deg)
  dis_ref[...] = jnp.broadcast_to(d, (BLK, F))
  for t in range(T):
    ux_ref[t] = x_ref[t] * d


_prep_call = pl.pallas_call(
    _prep_body,
    grid=(GRID,),
    in_specs=[
        pl.BlockSpec((NC, BLK, F), lambda i: (0, i, 0)),
        pl.BlockSpec((T, BLK, F), lambda i: (0, i, 0)),
    ],
    out_specs=[
        pl.BlockSpec((BLK, F), lambda i: (i, 0)),
        pl.BlockSpec((T, BLK, F), lambda i: (0, i, 0)),
    ],
    out_shape=[
        jax.ShapeDtypeStruct((N_PAD, F), jnp.float32),
        jax.ShapeDtypeStruct((T, N_PAD, F), jnp.float32),
    ],
)


def _make_cell(has_h):
  def body(*refs):
    if has_h:
      (sx_ref, ux_ref, sh_ref, uh_ref, c_ref, dis_ref, wx_ref, wh_ref, b_ref,
       h_o, c_o, uh_o) = refs
    else:
      (sx_ref, ux_ref, c_ref, dis_ref, wx_ref, b_ref, h_o, c_o, uh_o) = refs
    dis = dis_ref[...]
    px = dis * (sx_ref[0] + sx_ref[1] + ux_ref[...])
    cc = jnp.dot(px, wx_ref[...], preferred_element_type=jnp.float32)
    cc = cc + b_ref[...]
    if has_h:
      ph = dis * (sh_ref[0] + sh_ref[1] + uh_ref[...])
      cc = cc + jnp.dot(ph, wh_ref[...], preferred_element_type=jnp.float32)
    gi = jax.nn.sigmoid(cc[:, 0:H])
    gf = jax.nn.sigmoid(cc[:, H:2 * H])
    go = jax.nn.sigmoid(cc[:, 2 * H:3 * H])
    gg = jnp.tanh(cc[:, 3 * H:4 * H])
    c_new = gf * c_ref[...] + gi * gg
    h_new = go * jnp.tanh(c_new)
    h_o[...] = h_new
    c_o[...] = c_new
    uh_o[...] = dis * h_new

  part = pl.BlockSpec((NC, BLK, F), lambda i: (0, i, 0))
  full = pl.BlockSpec((BLK, F), lambda i: (i, 0))
  wspec = pl.BlockSpec((F, 4 * H), lambda i: (0, 0))
  bspec = pl.BlockSpec((1, 4 * H), lambda i: (0, 0))
  if has_h:
    in_specs = [part, full, part, full, full, full, wspec, wspec, bspec]
  else:
    in_specs = [part, full, full, full, wspec, bspec]
  return pl.pallas_call(
      body,
      grid=(GRID,),
      in_specs=in_specs,
      out_specs=[full, full, full],
      out_shape=[jax.ShapeDtypeStruct((N_PAD, F), jnp.float32)] * 3,
  )


_cell_h = _make_cell(True)
_cell_nh = _make_cell(False)


def kernel(x, edge_index, W0, b0, W1, b1):
  src = edge_index[0]
  dst = edge_index[1]
  pad = jnp.full((E_PAD - src.shape[0],), N, dtype=jnp.int32)
  src_w = jnp.concatenate([src, pad]).reshape(NC, NS, NCHUNK, CHUNK)
  dst_w = jnp.concatenate([dst, pad]).reshape(NC, NS, NCHUNK, CHUNK)

  zeros128 = jnp.zeros((N_PAD, F), jnp.float32)
  ones128 = jnp.ones((CHUNK, F), jnp.float32)

  hist = _hist_call(dst_w, ones128, zeros128)
  x_pad = jnp.pad(x[0], ((0, 0), (0, N_PAD - N), (0, 0)))
  dis, ux0 = _prep_call(hist, x_pad)

  b0r = b0.reshape(1, 4 * H)
  b1r = b1.reshape(1, 4 * H)
  params = [(W0[:F], W0[F:], b0r), (W1[:H], W1[H:], b1r)]

  ux = ux0
  h = c = None
  for layer in range(2):
    wx, wh, br = params[layer]
    sx_all = _prop4(ux, src_w, dst_w, zeros128)
    outs = []
    for t in range(T):
      if t == 0:
        c_prev = zeros128
        h, c, uh = _cell_nh(sx_all[t], ux[t], c_prev, dis, wx, br)
      else:
        sh = _prop1(uh[None], src_w, dst_w, zeros128)
        h, c, uh = _cell_h(sx_all[t], ux[t], sh[0], uh, c, dis, wx, wh, br)
      outs.append(uh)
    ux = jnp.stack(outs)

  return (h[:N][None], c[:N][None])
